# baseline (speedup 1.0000x reference)
import functools

import jax
import jax.numpy as jnp
from jax import lax
from jax.experimental import pallas as pl
from jax.experimental.pallas import tpu as pltpu
from jax.experimental.pallas import tpu_sc as plsc

N = 4096
D = 256
E = 65536
DH = D // 2

_NC = 2
_NS = 16
_NW = _NC * _NS
_EPW = E // _NW


def _sc_gather(s_tab, og_tab, r0, r1):
    gch = 128
    mesh = plsc.VectorSubcoreMesh(core_axis_name="c", subcore_axis_name="s")

    @functools.partial(
        pl.kernel, mesh=mesh,
        compiler_params=pltpu.CompilerParams(needs_layout_passes=False),
        out_type=[jax.ShapeDtypeStruct((E, D), jnp.float32),
                  jax.ShapeDtypeStruct((E, D), jnp.float32)],
        scratch_types=[pltpu.VMEM((_EPW,), jnp.int32),
                       pltpu.VMEM((_EPW,), jnp.int32),
                       pltpu.VMEM((gch, D), jnp.float32),
                       pltpu.VMEM((gch, D), jnp.float32),
                       pltpu.SemaphoreType.DMA,
                       pltpu.SemaphoreType.DMA],
    )
    def k(s_hbm, og_hbm, r0_hbm, r1_hbm, sg_hbm, ogg_hbm,
          i0, i1, buf0, buf1, sem0, sem1):
        wid = lax.axis_index("s") * _NC + lax.axis_index("c")
        base = wid * _EPW
        pltpu.sync_copy(r0_hbm.at[pl.ds(base, _EPW)], i0)
        pltpu.sync_copy(r1_hbm.at[pl.ds(base, _EPW)], i1)

        def body(c, carry):
            off = c * gch
            cp0 = pltpu.async_copy(s_hbm.at[i0.at[pl.ds(off, gch)]],
                                   buf0, sem0)
            cp1 = pltpu.async_copy(og_hbm.at[i1.at[pl.ds(off, gch)]],
                                   buf1, sem1)
            cp0.wait()
            cp1.wait()
            pltpu.sync_copy(buf0, sg_hbm.at[pl.ds(base + off, gch)])
            pltpu.sync_copy(buf1, ogg_hbm.at[pl.ds(base + off, gch)])
            return carry

        lax.fori_loop(0, _EPW // gch, body, 0)

    return k(s_tab, og_tab, r0, r1)


_RPW = N // _NW
_CAP = 3072
_CHB = 2048
_GB = 128
_SLAB = 8
_HALF = 0.5 * (N - 1)


def _iota16():
    return lax.iota(jnp.int32, 16)


def _splat(x, dtype=jnp.int32):
    return jax.lax.broadcast(x, (16,)).astype(dtype)


def _sigm(x):
    return 1.0 / (1.0 + jnp.exp(-x))


def _sc_edges_a(r0, r1, coeff, t3):
    mesh = plsc.VectorSubcoreMesh(core_axis_name="c", subcore_axis_name="s")

    @functools.partial(
        pl.kernel, mesh=mesh,
        compiler_params=pltpu.CompilerParams(needs_layout_passes=False),
        out_type=[jax.ShapeDtypeStruct((N * DH,), jnp.float32),
                  jax.ShapeDtypeStruct((N,), jnp.float32),
                  jax.ShapeDtypeStruct((_NW * _CAP,), jnp.int32),
                  jax.ShapeDtypeStruct((_NW * _CAP,), jnp.int32),
                  jax.ShapeDtypeStruct((_NW * _CAP,), jnp.float32)],
        scratch_types=[pltpu.VMEM((_CHB,), jnp.int32),
                       pltpu.VMEM((_CHB,), jnp.int32),
                       pltpu.VMEM((_CHB,), jnp.float32),
                       pltpu.VMEM((_CAP,), jnp.int32),
                       pltpu.VMEM((_CAP,), jnp.int32),
                       pltpu.VMEM((_CAP,), jnp.float32),
                       pltpu.VMEM((_CAP,), jnp.int32),
                       pltpu.VMEM((_CAP,), jnp.float32),
                       pltpu.VMEM((_SLAB * N,), jnp.float32),
                       pltpu.VMEM((_SLAB * N,), jnp.int32),
                       pltpu.VMEM((_RPW,), jnp.float32),
                       pltpu.VMEM((_GB, DH), jnp.float32),
                       pltpu.VMEM((_RPW * DH,), jnp.float32),
                       pltpu.SemaphoreType.DMA],
    )
    def k(r0_hbm, r1_hbm, cf_hbm, t3_hbm,
          c1_hbm, rs_hbm, ea_hbm, eb_hbm, ed_hbm,
          ch0, ch1, chc, sa, sb, sc, se, sd, slab_c, slab_e, rowd,
          gbuf, c1acc, sem):
        wid = lax.axis_index("s") * _NC + lax.axis_index("c")
        lo = wid * _RPW
        iota = _iota16()

        def init_body(i, _):
            z = jnp.zeros((16,), jnp.float32)
            mone = jnp.full((16,), -1, jnp.int32)
            slab_c[pl.ds(i * 16, 16)] = z
            slab_e[pl.ds(i * 16, 16)] = mone
            return 0
        lax.fori_loop(0, _SLAB * N // 16, init_body, 0)

        def init2(i, _):
            sd[pl.ds(i * 16, 16)] = jnp.zeros((16,), jnp.float32)
            sb[pl.ds(i * 16, 16)] = jnp.zeros((16,), jnp.int32)
            se[pl.ds(i * 16, 16)] = jnp.full((16,), -1, jnp.int32)
            sa[pl.ds(i * 16, 16)] = jnp.zeros((16,), jnp.int32)
            return 0
        lax.fori_loop(0, _CAP // 16, init2, 0)

        def init3(i, _):
            c1acc[pl.ds(i * 16, 16)] = jnp.zeros((16,), jnp.float32)
            return 0
        lax.fori_loop(0, _RPW * DH // 16, init3, 0)
        rowd[pl.ds(0, 16)] = jnp.zeros((16,), jnp.float32)

        def init4(i, _):
            rowd[pl.ds(i * 16, 16)] = jnp.zeros((16,), jnp.float32)
            return 0
        lax.fori_loop(0, _RPW // 16, init4, 0)

        def scan_chunk(cc, cnt):
            pltpu.sync_copy(r0_hbm.at[pl.ds(cc * _CHB, _CHB)], ch0)
            pltpu.sync_copy(r1_hbm.at[pl.ds(cc * _CHB, _CHB)], ch1)
            pltpu.sync_copy(cf_hbm.at[pl.ds(cc * _CHB, _CHB)], chc)

            def scan_batch(i, cnt):
                a = ch0[pl.ds(i * 16, 16)]
                al = a - lo
                m = (al >= 0) & (al < _RPW)
                nz = jnp.sum(m.astype(jnp.int32))

                @pl.when(nz > 0)
                def _():
                    pos = _splat(cnt) + jnp.cumsum(m.astype(jnp.int32)) - 1
                    b = ch1[pl.ds(i * 16, 16)]
                    cf = chc[pl.ds(i * 16, 16)]
                    eid = _splat(cc * _CHB + i * 16) + iota
                    plsc.store_scatter(sa, [pos], al, mask=m)
                    plsc.store_scatter(sb, [pos], b, mask=m)
                    plsc.store_scatter(sc, [pos], cf, mask=m)
                    plsc.store_scatter(se, [pos], eid, mask=m)
                return cnt + nz
            return lax.fori_loop(0, _CHB // 16, scan_batch, cnt)
        count = lax.fori_loop(0, E // _CHB, scan_chunk, jnp.int32(0))
        nbat = (count + 15) >> 4

        for sub in range(_RPW // _SLAB):
            def p1(i, _):
                al = sa[pl.ds(i * 16, 16)]
                e = se[pl.ds(i * 16, 16)]
                m = (e >= 0) & ((al >> 3) == sub)
                lidx = (al & (_SLAB - 1)) * N + sb[pl.ds(i * 16, 16)]

                @pl.when(jnp.sum(m.astype(jnp.int32)) > 0)
                def _():
                    plsc.addupdate_scatter(slab_c, [lidx],
                                           sc[pl.ds(i * 16, 16)], mask=m)
                    plsc.store_scatter(slab_e, [lidx], e, mask=m)
                return 0
            lax.fori_loop(0, nbat, p1, 0)

            def p2(i, _):
                al = sa[pl.ds(i * 16, 16)]
                b = sb[pl.ds(i * 16, 16)]
                e = se[pl.ds(i * 16, 16)]
                m = (e >= 0) & ((al >> 3) == sub)
                lidx = (al & (_SLAB - 1)) * N + b

                @pl.when(jnp.sum(m.astype(jnp.int32)) > 0)
                def _():
                    cs = plsc.load_gather(slab_c, [lidx], mask=m)
                    ce = plsc.load_gather(slab_e, [lidx], mask=m)
                    rep = m & (ce == e) & ((al + lo) != b)
                    delta = jnp.where(rep, _sigm(cs) - 0.5, 0.0)
                    sd[pl.ds(i * 16, 16)] = sd[pl.ds(i * 16, 16)] + delta
                    plsc.addupdate_scatter(rowd, [al], delta, mask=rep)
                    plsc.store_scatter(slab_c, [lidx],
                                       jnp.zeros((16,), jnp.float32), mask=m)
                    plsc.store_scatter(slab_e, [lidx],
                                       jnp.full((16,), -1, jnp.int32), mask=m)
                return 0
            lax.fori_loop(0, nbat, p2, 0)

        nch = (count + _GB - 1) >> 7

        def acc_chunk(g, _):
            cp = pltpu.async_copy(t3_hbm.at[sb.at[pl.ds(g * _GB, _GB)]],
                                  gbuf, sem)
            cp.wait()

            def acc_edge(j, _):
                i = g * _GB + j
                dsp = plsc.load_gather(sd, [_splat(i)])
                asp = plsc.load_gather(sa, [_splat(i)])
                for kk in range(DH // 16):
                    v = gbuf[j, pl.ds(kk * 16, 16)] * dsp
                    plsc.addupdate_scatter(c1acc,
                                           [asp * DH + kk * 16 + iota], v)
                return 0
            lax.fori_loop(0, _GB, acc_edge, 0)
            return 0
        lax.fori_loop(0, nch, acc_chunk, 0)

        def fin_batch(i, _):
            al = sa[pl.ds(i * 16, 16)]
            e = se[pl.ds(i * 16, 16)]
            valid = e >= 0
            rd = plsc.load_gather(rowd, [al], mask=valid)
            d2 = sd[pl.ds(i * 16, 16)] / (rd + _HALF)
            sd[pl.ds(i * 16, 16)] = jnp.where(valid, d2, 0.0)
            sa[pl.ds(i * 16, 16)] = al + lo
            sb[pl.ds(i * 16, 16)] = jnp.where(valid, sb[pl.ds(i * 16, 16)],
                                              -1)
            return 0
        lax.fori_loop(0, _CAP // 16, fin_batch, 0)

        def rs_batch(i, _):
            rowd[pl.ds(i * 16, 16)] = rowd[pl.ds(i * 16, 16)] + _HALF
            return 0
        lax.fori_loop(0, _RPW // 16, rs_batch, 0)

        pltpu.sync_copy(c1acc, c1_hbm.at[pl.ds(lo * DH, _RPW * DH)])
        pltpu.sync_copy(rowd, rs_hbm.at[pl.ds(lo, _RPW)])
        pltpu.sync_copy(sa, ea_hbm.at[pl.ds(wid * _CAP, _CAP)])
        pltpu.sync_copy(sb, eb_hbm.at[pl.ds(wid * _CAP, _CAP)])
        pltpu.sync_copy(sd, ed_hbm.at[pl.ds(wid * _CAP, _CAP)])

    return k(r0, r1, coeff, t3)


def _sc_edges_b(ea, eb, ed2, t3):
    mesh = plsc.VectorSubcoreMesh(core_axis_name="c", subcore_axis_name="s")
    tot = _NW * _CAP

    @functools.partial(
        pl.kernel, mesh=mesh,
        compiler_params=pltpu.CompilerParams(needs_layout_passes=False),
        out_type=jax.ShapeDtypeStruct((N * DH,), jnp.float32),
        scratch_types=[pltpu.VMEM((_CHB,), jnp.int32),
                       pltpu.VMEM((_CHB,), jnp.int32),
                       pltpu.VMEM((_CHB,), jnp.float32),
                       pltpu.VMEM((_CAP,), jnp.int32),
                       pltpu.VMEM((_CAP,), jnp.int32),
                       pltpu.VMEM((_CAP,), jnp.float32),
                       pltpu.VMEM((_GB, DH), jnp.float32),
                       pltpu.VMEM((_RPW * DH,), jnp.float32),
                       pltpu.SemaphoreType.DMA],
    )
    def k(ea_hbm, eb_hbm, ed_hbm, t3_hbm, c2_hbm,
          cha, chb, chd, sa, sbl, sd2, gbuf, c2acc, sem):
        wid = lax.axis_index("s") * _NC + lax.axis_index("c")
        lo = wid * _RPW
        iota = _iota16()

        def init_body(i, _):
            c2acc[pl.ds(i * 16, 16)] = jnp.zeros((16,), jnp.float32)
            return 0
        lax.fori_loop(0, _RPW * DH // 16, init_body, 0)

        def init2(i, _):
            sa[pl.ds(i * 16, 16)] = jnp.zeros((16,), jnp.int32)
            sbl[pl.ds(i * 16, 16)] = jnp.zeros((16,), jnp.int32)
            sd2[pl.ds(i * 16, 16)] = jnp.zeros((16,), jnp.float32)
            return 0
        lax.fori_loop(0, _CAP // 16, init2, 0)

        def scan_chunk(cc, cnt):
            pltpu.sync_copy(ea_hbm.at[pl.ds(cc * _CHB, _CHB)], cha)
            pltpu.sync_copy(eb_hbm.at[pl.ds(cc * _CHB, _CHB)], chb)
            pltpu.sync_copy(ed_hbm.at[pl.ds(cc * _CHB, _CHB)], chd)

            def scan_batch(i, cnt):
                b = chb[pl.ds(i * 16, 16)]
                bl = b - lo
                m = (bl >= 0) & (bl < _RPW)
                nz = jnp.sum(m.astype(jnp.int32))

                @pl.when(nz > 0)
                def _():
                    pos = _splat(cnt) + jnp.cumsum(m.astype(jnp.int32)) - 1
                    plsc.store_scatter(sa, [pos], cha[pl.ds(i * 16, 16)],
                                       mask=m)
                    plsc.store_scatter(sbl, [pos], bl, mask=m)
                    plsc.store_scatter(sd2, [pos], chd[pl.ds(i * 16, 16)],
                                       mask=m)
                return cnt + nz
            return lax.fori_loop(0, _CHB // 16, scan_batch, cnt)
        count = lax.fori_loop(0, tot // _CHB, scan_chunk, jnp.int32(0))

        nch = (count + _GB - 1) >> 7

        def acc_chunk(g, _):
            cp = pltpu.async_copy(t3_hbm.at[sa.at[pl.ds(g * _GB, _GB)]],
                                  gbuf, sem)
            cp.wait()

            def acc_edge(j, _):
                i = g * _GB + j
                dsp = plsc.load_gather(sd2, [_splat(i)])
                bsp = plsc.load_gather(sbl, [_splat(i)])
                for kk in range(DH // 16):
                    v = gbuf[j, pl.ds(kk * 16, 16)] * dsp
                    plsc.addupdate_scatter(c2acc,
                                           [bsp * DH + kk * 16 + iota], v)
                return 0
            lax.fori_loop(0, _GB, acc_edge, 0)
            return 0
        lax.fori_loop(0, nch, acc_chunk, 0)

        pltpu.sync_copy(c2acc, c2_hbm.at[pl.ds(lo * DH, _RPW * DH)])

    return k(ea, eb, ed2, t3)


def _dotT(x, w):
    return lax.dot_general(x, w, (((1,), (1,)), ((), ())),
                           preferred_element_type=jnp.float32)


def _prep_body(obj_ref, ws_ref, wsb_ref, wo_ref, wob_ref, ww_ref, wt3_ref,
               wt3b_ref, s_ref, og_ref, t3_ref):
    x = obj_ref[...]
    s_ref[...] = _dotT(x, ws_ref[...]) + wsb_ref[...]
    og_ref[...] = (_dotT(x, wo_ref[...]) + wob_ref[...]) * ww_ref[...]
    t3_ref[...] = jnp.maximum(_dotT(x, wt3_ref[...]) + wt3b_ref[...], 0.0)


def _prep(obj, ws_w, ws_b, wo_w, wo_b, w_w, wt3_w, wt3_b):
    blk = 512
    grid = N // blk
    full = lambda shape: pl.BlockSpec(shape, lambda i: (0, 0))
    return pl.pallas_call(
        _prep_body,
        grid=(grid,),
        in_specs=[
            pl.BlockSpec((blk, D), lambda i: (i, 0)),
            full((D, D)), full((1, D)),
            full((D, D)), full((1, D)),
            full((1, D)),
            full((DH, D)), full((1, DH)),
        ],
        out_specs=[
            pl.BlockSpec((blk, D), lambda i: (i, 0)),
            pl.BlockSpec((blk, D), lambda i: (i, 0)),
            pl.BlockSpec((blk, DH), lambda i: (i, 0)),
        ],
        out_shape=[
            jax.ShapeDtypeStruct((N, D), jnp.float32),
            jax.ShapeDtypeStruct((N, D), jnp.float32),
            jax.ShapeDtypeStruct((N, DH), jnp.float32),
        ],
    )(obj, ws_w, ws_b.reshape(1, D), wo_w, wo_b.reshape(1, D),
      w_w.reshape(1, D), wt3_w, wt3_b.reshape(1, DH))


def _coeff_body(union_ref, sg_ref, og_ref, wu_ref, wub_ref, wb_ref, out_ref):
    u = _dotT(union_ref[...], wu_ref[...]) + wub_ref[...]
    p = sg_ref[...] * og_ref[...] * u
    rows = out_ref.shape[0]
    out_ref[...] = (jnp.sum(p, axis=1) + wb_ref[0, 0]).reshape(rows, 256)


def _coeff(union, sg, og, wu_w, wu_b, w_b):
    blk = 4096
    grid = E // blk
    rows = blk // 256
    full = lambda shape: pl.BlockSpec(shape, lambda i: (0, 0))
    out = pl.pallas_call(
        _coeff_body,
        grid=(grid,),
        in_specs=[
            pl.BlockSpec((blk, D), lambda i: (i, 0)),
            pl.BlockSpec((blk, D), lambda i: (i, 0)),
            pl.BlockSpec((blk, D), lambda i: (i, 0)),
            full((D, D)), full((1, D)), full((1, D)),
        ],
        out_specs=pl.BlockSpec((rows, 256), lambda i: (i, 0)),
        out_shape=jax.ShapeDtypeStruct((E // 256, 256), jnp.float32),
    )(union, sg, og, wu_w, wu_b.reshape(1, D),
      jnp.broadcast_to(w_b.reshape(1, 1), (1, D)))
    return out.reshape(E)


def _mlp_body(obj_ref, t3full_ref, rsfull_ref, c1_ref, c2_ref, t1_ref,
              t1b_ref, lng_ref, lnb_ref, t2_ref, t2b_ref, out_ref):
    i = pl.program_id(0)
    blk = obj_ref.shape[0]
    t3full = t3full_ref[...]
    rsfull = rsfull_ref[...]
    csum1 = jnp.sum(t3full, axis=0, keepdims=True)
    csum2 = jnp.sum(t3full / rsfull, axis=0, keepdims=True)
    t3_blk = t3full_ref[pl.ds(i * blk, blk), :]
    rs_blk = rsfull_ref[pl.ds(i * blk, blk), :]
    ctx1 = (0.5 * csum1 - 0.5 * t3_blk + c1_ref[...]) / rs_blk
    ctx2 = 0.5 * csum2 - 0.5 * t3_blk / rs_blk + c2_ref[...]
    ctx = jnp.concatenate([ctx1, ctx2], axis=1)
    h = _dotT(ctx, t1_ref[...]) + t1b_ref[...]
    mu = jnp.mean(h, axis=1, keepdims=True)
    dh = h - mu
    var = jnp.mean(dh * dh, axis=1, keepdims=True)
    h = dh * lax.rsqrt(var + 1e-5) * lng_ref[...] + lnb_ref[...]
    h = jnp.maximum(h, 0.0)
    nb = _dotT(h, t2_ref[...]) + t2b_ref[...]
    out_ref[...] = jnp.maximum(obj_ref[...] + nb, 0.0)


def _mlp(obj, t3, rowsum, c1, c2, t1_w, t1_b, ln_g, ln_b, t2_w, t2_b):
    blk = 512
    grid = N // blk
    full = lambda shape: pl.BlockSpec(shape, lambda i: (0, 0))
    Dq = D // 4
    return pl.pallas_call(
        _mlp_body,
        grid=(grid,),
        in_specs=[
            pl.BlockSpec((blk, D), lambda i: (i, 0)),
            full((N, DH)), full((N, 1)),
            pl.BlockSpec((blk, DH), lambda i: (i, 0)),
            pl.BlockSpec((blk, DH), lambda i: (i, 0)),
            full((Dq, D)), full((1, Dq)), full((1, Dq)), full((1, Dq)),
            full((D, Dq)), full((1, D)),
        ],
        out_specs=pl.BlockSpec((blk, D), lambda i: (i, 0)),
        out_shape=jax.ShapeDtypeStruct((N, D), jnp.float32),
    )(obj, t3, rowsum.reshape(N, 1), c1, c2, t1_w, t1_b.reshape(1, Dq),
      ln_g.reshape(1, Dq), ln_b.reshape(1, Dq), t2_w, t2_b.reshape(1, D))


def kernel(obj_feats, union_feats, rel_pair_idx, ws_w, ws_b, wo_w, wo_b,
           wu_w, wu_b, w_w, w_b, wt3_w, wt3_b, t1_w, t1_b, ln_g, ln_b,
           t2_w, t2_b):
    r0 = rel_pair_idx[:, 0].astype(jnp.int32)
    r1 = rel_pair_idx[:, 1].astype(jnp.int32)

    s_tab, og_tab, t3 = _prep(obj_feats, ws_w, ws_b, wo_w, wo_b, w_w,
                              wt3_w, wt3_b)

    sg, og = _sc_gather(s_tab, og_tab, r0, r1)

    coeff = _coeff(union_feats, sg, og, wu_w, wu_b, w_b)

    c1f, rowsum, ea, eb, ed2 = _sc_edges_a(r0, r1, coeff, t3)
    c2f = _sc_edges_b(ea, eb, ed2, t3)
    c1 = c1f.reshape(N, DH)
    c2 = c2f.reshape(N, DH)

    return _mlp(obj_feats, t3, rowsum, c1, c2, t1_w, t1_b, ln_g, ln_b,
                t2_w, t2_b)


# speedup vs baseline: 1.3132x; 1.3132x over previous
import functools

import jax
import jax.numpy as jnp
from jax import lax
from jax.experimental import pallas as pl
from jax.experimental.pallas import tpu as pltpu
from jax.experimental.pallas import tpu_sc as plsc

N = 4096
D = 256
E = 65536
DH = D // 2

def _dotT(x, w):
    return lax.dot_general(x, w, (((1,), (1,)), ((), ())),
                           preferred_element_type=jnp.float32)


_NC = 2
_NS = 16
_NW = _NC * _NS
_EPW = E // _NW


_RPW = N // _NW
_CAP = 3072
_CHB = 1024
_GB = 64
_GBS = 6
_SLAB = 8
_HALF = 0.5 * (N - 1)

_SC_PARAMS = dict(
    compiler_params=pltpu.CompilerParams(needs_layout_passes=False))


def _iota16():
    return lax.iota(jnp.int32, 16)


def _splat(x, dtype=jnp.int32):
    return jax.lax.broadcast(x, (16,)).astype(dtype)


def _sigm(x):
    return 1.0 / (1.0 + jnp.exp(-x))


def _mesh():
    return plsc.VectorSubcoreMesh(core_axis_name="c", subcore_axis_name="s")


def _sc_gather(s_tab, og_tab, r0, r1):
    gch = 64
    nchunk = _EPW // gch

    @functools.partial(
        pl.kernel, mesh=_mesh(), **_SC_PARAMS,
        out_type=[jax.ShapeDtypeStruct((E, D), jnp.float32),
                  jax.ShapeDtypeStruct((E, D), jnp.float32)],
        scratch_types=[pltpu.VMEM((_EPW,), jnp.int32),
                       pltpu.VMEM((_EPW,), jnp.int32),
                       pltpu.VMEM((2, gch, D), jnp.float32),
                       pltpu.VMEM((2, gch, D), jnp.float32),
                       pltpu.SemaphoreType.DMA((2,)),
                       pltpu.SemaphoreType.DMA((2,)),
                       pltpu.SemaphoreType.DMA((2,)),
                       pltpu.SemaphoreType.DMA((2,))],
    )
    def k(s_hbm, og_hbm, r0_hbm, r1_hbm, sg_hbm, ogg_hbm,
          i0, i1, sbuf, obuf, gs, go, os_, oo):
        wid = lax.axis_index("s") * _NC + lax.axis_index("c")
        base = wid * _EPW
        pltpu.sync_copy(r0_hbm.at[pl.ds(base, _EPW)], i0)
        pltpu.sync_copy(r1_hbm.at[pl.ds(base, _EPW)], i1)

        def g_cp(c, p):
            off = c * gch
            return (
                pltpu.make_async_copy(s_hbm.at[i0.at[pl.ds(off, gch)]],
                                      sbuf.at[p], gs.at[p]),
                pltpu.make_async_copy(og_hbm.at[i1.at[pl.ds(off, gch)]],
                                      obuf.at[p], go.at[p]))

        def o_cp(c, p):
            off = base + c * gch
            return (
                pltpu.make_async_copy(sbuf.at[p],
                                      sg_hbm.at[pl.ds(off, gch)], os_.at[p]),
                pltpu.make_async_copy(obuf.at[p],
                                      ogg_hbm.at[pl.ds(off, gch)], oo.at[p]))

        for cp in g_cp(0, 0):
            cp.start()
        for c in range(nchunk):
            p = c & 1
            if c + 1 < nchunk:
                if c >= 1:
                    for cp in o_cp(c - 1, 1 - p):
                        cp.wait()
                for cp in g_cp(c + 1, 1 - p):
                    cp.start()
            for cp in g_cp(c, p):
                cp.wait()
            for cp in o_cp(c, p):
                cp.start()
        for cp in o_cp(nchunk - 2, (nchunk - 2) & 1):
            cp.wait()
        for cp in o_cp(nchunk - 1, (nchunk - 1) & 1):
            cp.wait()

    return k(s_tab, og_tab, r0, r1)


def _sc_edges_a(key, coeff, t3, zf32, zi32, mi32):
    nchunk = E // _CHB

    @functools.partial(
        pl.kernel, mesh=_mesh(), **_SC_PARAMS,
        out_type=[jax.ShapeDtypeStruct((N * DH,), jnp.float32),
                  jax.ShapeDtypeStruct((N,), jnp.float32),
                  jax.ShapeDtypeStruct((_NW * _CAP,), jnp.int32),
                  jax.ShapeDtypeStruct((_NW * _CAP,), jnp.float32)],
        scratch_types=[pltpu.VMEM((2, _CHB), jnp.int32),
                       pltpu.VMEM((2, _CHB), jnp.float32),
                       pltpu.VMEM((_CAP,), jnp.int32),
                       pltpu.VMEM((_CAP,), jnp.int32),
                       pltpu.VMEM((_CAP,), jnp.float32),
                       pltpu.VMEM((_CAP,), jnp.int32),
                       pltpu.VMEM((_CAP,), jnp.float32),
                       pltpu.VMEM((_SLAB * N,), jnp.float32),
                       pltpu.VMEM((_SLAB * N,), jnp.int32),
                       pltpu.VMEM((_RPW,), jnp.float32),
                       pltpu.VMEM((2, _GB, DH), jnp.float32),
                       pltpu.VMEM((_RPW * DH,), jnp.float32),
                       pltpu.SemaphoreType.DMA((2,)),
                       pltpu.SemaphoreType.DMA((2,)),
                       pltpu.SemaphoreType.DMA((2,))],
    )
    def k(key_hbm, cf_hbm, t3_hbm, zf_hbm, zi_hbm, mi_hbm,
          c1_hbm, rs_hbm, eba_hbm, ed_hbm,
          kb, cb, sa, sb, sc, se, sd, slab_c, slab_e, rowd,
          gbuf, c1acc, ksem, csem, gsem):
        wid = lax.axis_index("s") * _NC + lax.axis_index("c")
        lo = wid * _RPW
        klo = lo * N
        khi = (lo + _RPW) * N
        iota = _iota16()

        def ch_cp(c, p):
            return (
                pltpu.make_async_copy(key_hbm.at[pl.ds(c * _CHB, _CHB)],
                                      kb.at[p], ksem.at[p]),
                pltpu.make_async_copy(cf_hbm.at[pl.ds(c * _CHB, _CHB)],
                                      cb.at[p], csem.at[p]))

        for cp in ch_cp(0, 0):
            cp.start()

        pltpu.sync_copy(zf_hbm, slab_c)
        pltpu.sync_copy(zf_hbm.at[pl.ds(0, _RPW * DH)], c1acc)
        pltpu.sync_copy(zf_hbm.at[pl.ds(0, _CAP)], sd)
        pltpu.sync_copy(zf_hbm.at[pl.ds(0, _RPW)], rowd)
        pltpu.sync_copy(zi_hbm, sa)
        pltpu.sync_copy(zi_hbm, sb)
        pltpu.sync_copy(mi_hbm, se)

        cnt = jnp.int32(0)
        for c in range(nchunk):
            p = c & 1
            if c + 1 < nchunk:
                for cp in ch_cp(c + 1, 1 - p):
                    cp.start()
            for cp in ch_cp(c, p):
                cp.wait()

            def scan_batch(i, cnt, _c=c, _p=p):
                kv = kb[_p, pl.ds(i * 16, 16)]
                m = (kv >= klo) & (kv < khi)
                nz = jnp.sum(m.astype(jnp.int32))

                @pl.when(nz > 0)
                def _():
                    pos = _splat(cnt) + jnp.cumsum(m.astype(jnp.int32)) - 1
                    al = (kv >> 12) - lo
                    b = kv & (N - 1)
                    cf = cb[_p, pl.ds(i * 16, 16)]
                    eid = _splat(_c * _CHB + i * 16) + iota
                    plsc.store_scatter(sa, [pos], al, mask=m)
                    plsc.store_scatter(sb, [pos], b, mask=m)
                    plsc.store_scatter(sc, [pos], cf, mask=m)
                    plsc.store_scatter(se, [pos], eid, mask=m)
                return cnt + nz
            cnt = lax.fori_loop(0, _CHB // 16, scan_batch, cnt)
        count = cnt
        nbat = (count + 15) >> 4

        for sub in range(_RPW // _SLAB):
            def p1(i, _):
                al = sa[pl.ds(i * 16, 16)]
                e = se[pl.ds(i * 16, 16)]
                m = (e >= 0) & ((al >> 3) == sub)
                lidx = (al & (_SLAB - 1)) * N + sb[pl.ds(i * 16, 16)]

                @pl.when(jnp.sum(m.astype(jnp.int32)) > 0)
                def _():
                    plsc.addupdate_scatter(slab_c, [lidx],
                                           sc[pl.ds(i * 16, 16)], mask=m)
                    plsc.store_scatter(slab_e, [lidx], e, mask=m)
                return 0
            lax.fori_loop(0, nbat, p1, 0)

            def p2(i, _):
                al = sa[pl.ds(i * 16, 16)]
                b = sb[pl.ds(i * 16, 16)]
                e = se[pl.ds(i * 16, 16)]
                m = (e >= 0) & ((al >> 3) == sub)
                lidx = (al & (_SLAB - 1)) * N + b

                @pl.when(jnp.sum(m.astype(jnp.int32)) > 0)
                def _():
                    cs = plsc.load_gather(slab_c, [lidx], mask=m)
                    ce = plsc.load_gather(slab_e, [lidx], mask=m)
                    rep = m & (ce == e) & ((al + lo) != b)
                    delta = jnp.where(rep, _sigm(cs) - 0.5, 0.0)
                    sd[pl.ds(i * 16, 16)] = sd[pl.ds(i * 16, 16)] + delta
                    plsc.addupdate_scatter(rowd, [al], delta, mask=rep)
                    plsc.store_scatter(slab_c, [lidx],
                                       jnp.zeros((16,), jnp.float32), mask=m)
                return 0
            lax.fori_loop(0, nbat, p2, 0)

        nch = (count + _GB - 1) >> _GBS

        def g_start(cs_, p):
            pltpu.make_async_copy(
                t3_hbm.at[sb.at[pl.ds(cs_ * _GB, _GB)]],
                gbuf.at[p], gsem.at[p]).start()

        def g_wait(cs_, p):
            pltpu.make_async_copy(
                t3_hbm.at[sb.at[pl.ds(cs_ * _GB, _GB)]],
                gbuf.at[p], gsem.at[p]).wait()

        def proc(cs_, p):
            def acc_edge(j, _):
                i = cs_ * _GB + j
                dsp = plsc.load_gather(sd, [_splat(i)])
                asp = plsc.load_gather(sa, [_splat(i)])
                for kk in range(DH // 16):
                    v = gbuf[p, j, pl.ds(kk * 16, 16)] * dsp
                    plsc.addupdate_scatter(c1acc,
                                           [asp * DH + kk * 16 + iota], v)
                return 0
            lax.fori_loop(0, _GB, acc_edge, 0)

        @pl.when(nch > 0)
        def _():
            g_start(0, 0)

        def pair(i, _):
            c0 = 2 * i

            @pl.when(c0 + 1 < nch)
            def _():
                g_start(c0 + 1, 1)
            g_wait(c0, 0)
            proc(c0, 0)

            @pl.when(c0 + 2 < nch)
            def _():
                g_start(c0 + 2, 0)

            @pl.when(c0 + 1 < nch)
            def _():
                g_wait(c0 + 1, 1)
                proc(c0 + 1, 1)
            return 0
        lax.fori_loop(0, (nch + 1) >> 1, pair, 0)

        def fin_batch(i, _):
            al = sa[pl.ds(i * 16, 16)]
            e = se[pl.ds(i * 16, 16)]
            valid = e >= 0
            rd = plsc.load_gather(rowd, [al], mask=valid)
            d2 = sd[pl.ds(i * 16, 16)] / (rd + _HALF)
            sd[pl.ds(i * 16, 16)] = jnp.where(valid, d2, 0.0)
            pb = (sb[pl.ds(i * 16, 16)] << 12) | (al + lo)
            sa[pl.ds(i * 16, 16)] = jnp.where(valid, pb, -1)
            return 0
        lax.fori_loop(0, _CAP // 16, fin_batch, 0)

        def rs_batch(i, _):
            rowd[pl.ds(i * 16, 16)] = rowd[pl.ds(i * 16, 16)] + _HALF
            return 0
        lax.fori_loop(0, _RPW // 16, rs_batch, 0)

        pltpu.sync_copy(c1acc, c1_hbm.at[pl.ds(lo * DH, _RPW * DH)])
        pltpu.sync_copy(rowd, rs_hbm.at[pl.ds(lo, _RPW)])
        pltpu.sync_copy(sa, eba_hbm.at[pl.ds(wid * _CAP, _CAP)])
        pltpu.sync_copy(sd, ed_hbm.at[pl.ds(wid * _CAP, _CAP)])

    return k(key, coeff, t3, zf32, zi32, mi32)


def _sc_edges_b(eba, ed2, t3, zf32, zi32):
    tot = _NW * _CAP
    nchunk = tot // _CHB

    @functools.partial(
        pl.kernel, mesh=_mesh(), **_SC_PARAMS,
        out_type=jax.ShapeDtypeStruct((N * DH,), jnp.float32),
        scratch_types=[pltpu.VMEM((2, _CHB), jnp.int32),
                       pltpu.VMEM((2, _CHB), jnp.float32),
                       pltpu.VMEM((_CAP,), jnp.int32),
                       pltpu.VMEM((_CAP,), jnp.int32),
                       pltpu.VMEM((_CAP,), jnp.float32),
                       pltpu.VMEM((2, _GB, DH), jnp.float32),
                       pltpu.VMEM((_RPW * DH,), jnp.float32),
                       pltpu.SemaphoreType.DMA((2,)),
                       pltpu.SemaphoreType.DMA((2,)),
                       pltpu.SemaphoreType.DMA((2,))],
    )
    def k(eba_hbm, ed_hbm, t3_hbm, zf_hbm, zi_hbm, c2_hbm,
          kb, db, sga, sbl, sd2, gbuf, c2acc, ksem, dsem, gsem):
        wid = lax.axis_index("s") * _NC + lax.axis_index("c")
        lo = wid * _RPW
        klo = lo << 12
        khi = (lo + _RPW) << 12
        iota = _iota16()

        def ch_cp(c, p):
            return (
                pltpu.make_async_copy(eba_hbm.at[pl.ds(c * _CHB, _CHB)],
                                      kb.at[p], ksem.at[p]),
                pltpu.make_async_copy(ed_hbm.at[pl.ds(c * _CHB, _CHB)],
                                      db.at[p], dsem.at[p]))

        for cp in ch_cp(0, 0):
            cp.start()
        pltpu.sync_copy(zf_hbm.at[pl.ds(0, _RPW * DH)], c2acc)
        pltpu.sync_copy(zf_hbm.at[pl.ds(0, _CAP)], sd2)
        pltpu.sync_copy(zi_hbm, sga)
        pltpu.sync_copy(zi_hbm, sbl)

        cnt = jnp.int32(0)
        for c in range(nchunk):
            p = c & 1
            if c + 1 < nchunk:
                for cp in ch_cp(c + 1, 1 - p):
                    cp.start()
            for cp in ch_cp(c, p):
                cp.wait()

            def scan_batch(i, cnt, _p=p):
                pb = kb[_p, pl.ds(i * 16, 16)]
                m = (pb >= klo) & (pb < khi)
                nz = jnp.sum(m.astype(jnp.int32))

                @pl.when(nz > 0)
                def _():
                    pos = _splat(cnt) + jnp.cumsum(m.astype(jnp.int32)) - 1
                    bl = (pb >> 12) - lo
                    a = pb & (N - 1)
                    d2 = db[_p, pl.ds(i * 16, 16)]
                    plsc.store_scatter(sga, [pos], a, mask=m)
                    plsc.store_scatter(sbl, [pos], bl, mask=m)
                    plsc.store_scatter(sd2, [pos], d2, mask=m)
                return cnt + nz
            cnt = lax.fori_loop(0, _CHB // 16, scan_batch, cnt)
        count = cnt
        nch = (count + _GB - 1) >> _GBS

        def g_start(cs_, p):
            pltpu.make_async_copy(
                t3_hbm.at[sga.at[pl.ds(cs_ * _GB, _GB)]],
                gbuf.at[p], gsem.at[p]).start()

        def g_wait(cs_, p):
            pltpu.make_async_copy(
                t3_hbm.at[sga.at[pl.ds(cs_ * _GB, _GB)]],
                gbuf.at[p], gsem.at[p]).wait()

        def proc(cs_, p):
            def acc_edge(j, _):
                i = cs_ * _GB + j
                dsp = plsc.load_gather(sd2, [_splat(i)])
                bsp = plsc.load_gather(sbl, [_splat(i)])
                for kk in range(DH // 16):
                    v = gbuf[p, j, pl.ds(kk * 16, 16)] * dsp
                    plsc.addupdate_scatter(c2acc,
                                           [bsp * DH + kk * 16 + iota], v)
                return 0
            lax.fori_loop(0, _GB, acc_edge, 0)

        @pl.when(nch > 0)
        def _():
            g_start(0, 0)

        def pair(i, _):
            c0 = 2 * i

            @pl.when(c0 + 1 < nch)
            def _():
                g_start(c0 + 1, 1)
            g_wait(c0, 0)
            proc(c0, 0)

            @pl.when(c0 + 2 < nch)
            def _():
                g_start(c0 + 2, 0)

            @pl.when(c0 + 1 < nch)
            def _():
                g_wait(c0 + 1, 1)
                proc(c0 + 1, 1)
            return 0
        lax.fori_loop(0, (nch + 1) >> 1, pair, 0)

        pltpu.sync_copy(c2acc, c2_hbm.at[pl.ds(lo * DH, _RPW * DH)])

    return k(eba, ed2, t3, zf32, zi32)


def _prep_body(obj_ref, ws_ref, wsb_ref, wo_ref, wob_ref, ww_ref, wt3_ref,
               wt3b_ref, s_ref, og_ref, t3_ref):
    x = obj_ref[...]
    s_ref[...] = _dotT(x, ws_ref[...]) + wsb_ref[...]
    og_ref[...] = (_dotT(x, wo_ref[...]) + wob_ref[...]) * ww_ref[...]
    t3_ref[...] = jnp.maximum(_dotT(x, wt3_ref[...]) + wt3b_ref[...], 0.0)


def _prep(obj, ws_w, ws_b, wo_w, wo_b, w_w, wt3_w, wt3_b):
    blk = 512
    grid = N // blk
    full = lambda shape: pl.BlockSpec(shape, lambda i: (0, 0))
    return pl.pallas_call(
        _prep_body,
        grid=(grid,),
        in_specs=[
            pl.BlockSpec((blk, D), lambda i: (i, 0)),
            full((D, D)), full((1, D)),
            full((D, D)), full((1, D)),
            full((1, D)),
            full((DH, D)), full((1, DH)),
        ],
        out_specs=[
            pl.BlockSpec((blk, D), lambda i: (i, 0)),
            pl.BlockSpec((blk, D), lambda i: (i, 0)),
            pl.BlockSpec((blk, DH), lambda i: (i, 0)),
        ],
        out_shape=[
            jax.ShapeDtypeStruct((N, D), jnp.float32),
            jax.ShapeDtypeStruct((N, D), jnp.float32),
            jax.ShapeDtypeStruct((N, DH), jnp.float32),
        ],
    )(obj, ws_w, ws_b.reshape(1, D), wo_w, wo_b.reshape(1, D),
      w_w.reshape(1, D), wt3_w, wt3_b.reshape(1, DH))


def _coeff_body(union_ref, sg_ref, og_ref, wu_ref, wub_ref, wb_ref, out_ref):
    u = _dotT(union_ref[...], wu_ref[...]) + wub_ref[...]
    p = sg_ref[...] * og_ref[...] * u
    rows = out_ref.shape[0]
    out_ref[...] = (jnp.sum(p, axis=1) + wb_ref[0, 0]).reshape(rows, 256)


def _coeff(union, sg, og, wu_w, wu_b, w_b):
    blk = 4096
    grid = E // blk
    rows = blk // 256
    full = lambda shape: pl.BlockSpec(shape, lambda i: (0, 0))
    out = pl.pallas_call(
        _coeff_body,
        grid=(grid,),
        in_specs=[
            pl.BlockSpec((blk, D), lambda i: (i, 0)),
            pl.BlockSpec((blk, D), lambda i: (i, 0)),
            pl.BlockSpec((blk, D), lambda i: (i, 0)),
            full((D, D)), full((1, D)), full((1, D)),
        ],
        out_specs=pl.BlockSpec((rows, 256), lambda i: (i, 0)),
        out_shape=jax.ShapeDtypeStruct((E // 256, 256), jnp.float32),
    )(union, sg, og, wu_w, wu_b.reshape(1, D),
      jnp.broadcast_to(w_b.reshape(1, 1), (1, D)))
    return out.reshape(E)


def _mlp_body(obj_ref, t3full_ref, rsfull_ref, c1_ref, c2_ref, t1_ref,
              t1b_ref, lng_ref, lnb_ref, t2_ref, t2b_ref, out_ref):
    i = pl.program_id(0)
    blk = obj_ref.shape[0]
    t3full = t3full_ref[...]
    rsfull = rsfull_ref[...]
    csum1 = jnp.sum(t3full, axis=0, keepdims=True)
    csum2 = jnp.sum(t3full / rsfull, axis=0, keepdims=True)
    t3_blk = t3full_ref[pl.ds(i * blk, blk), :]
    rs_blk = rsfull_ref[pl.ds(i * blk, blk), :]
    ctx1 = (0.5 * csum1 - 0.5 * t3_blk + c1_ref[...]) / rs_blk
    ctx2 = 0.5 * csum2 - 0.5 * t3_blk / rs_blk + c2_ref[...]
    ctx = jnp.concatenate([ctx1, ctx2], axis=1)
    h = _dotT(ctx, t1_ref[...]) + t1b_ref[...]
    mu = jnp.mean(h, axis=1, keepdims=True)
    dh = h - mu
    var = jnp.mean(dh * dh, axis=1, keepdims=True)
    h = dh * lax.rsqrt(var + 1e-5) * lng_ref[...] + lnb_ref[...]
    h = jnp.maximum(h, 0.0)
    nb = _dotT(h, t2_ref[...]) + t2b_ref[...]
    out_ref[...] = jnp.maximum(obj_ref[...] + nb, 0.0)


def _mlp(obj, t3, rowsum, c1, c2, t1_w, t1_b, ln_g, ln_b, t2_w, t2_b):
    blk = 512
    grid = N // blk
    full = lambda shape: pl.BlockSpec(shape, lambda i: (0, 0))
    Dq = D // 4
    return pl.pallas_call(
        _mlp_body,
        grid=(grid,),
        in_specs=[
            pl.BlockSpec((blk, D), lambda i: (i, 0)),
            full((N, DH)), full((N, 1)),
            pl.BlockSpec((blk, DH), lambda i: (i, 0)),
            pl.BlockSpec((blk, DH), lambda i: (i, 0)),
            full((Dq, D)), full((1, Dq)), full((1, Dq)), full((1, Dq)),
            full((D, Dq)), full((1, D)),
        ],
        out_specs=pl.BlockSpec((blk, D), lambda i: (i, 0)),
        out_shape=jax.ShapeDtypeStruct((N, D), jnp.float32),
    )(obj, t3, rowsum.reshape(N, 1), c1, c2, t1_w, t1_b.reshape(1, Dq),
      ln_g.reshape(1, Dq), ln_b.reshape(1, Dq), t2_w, t2_b.reshape(1, D))


def kernel(obj_feats, union_feats, rel_pair_idx, ws_w, ws_b, wo_w, wo_b,
           wu_w, wu_b, w_w, w_b, wt3_w, wt3_b, t1_w, t1_b, ln_g, ln_b,
           t2_w, t2_b):
    r0 = rel_pair_idx[:, 0].astype(jnp.int32)
    r1 = rel_pair_idx[:, 1].astype(jnp.int32)

    s_tab, og_tab, t3 = _prep(obj_feats, ws_w, ws_b, wo_w, wo_b, w_w,
                              wt3_w, wt3_b)

    sg, og = _sc_gather(s_tab, og_tab, r0, r1)

    coeff = _coeff(union_feats, sg, og, wu_w, wu_b, w_b)

    key = r0 * N + r1
    zf32 = jnp.zeros((_SLAB * N,), jnp.float32)
    zi32 = jnp.zeros((_CAP,), jnp.int32)
    mi32 = jnp.full((_CAP,), -1, jnp.int32)
    c1f, rowsum, eba, ed2 = _sc_edges_a(key, coeff, t3, zf32, zi32, mi32)
    c2f = _sc_edges_b(eba, ed2, t3, zf32, zi32)
    c1 = c1f.reshape(N, DH)
    c2 = c2f.reshape(N, DH)

    return _mlp(obj_feats, t3, rowsum, c1, c2, t1_w, t1_b, ln_g, ln_b,
                t2_w, t2_b)


# speedup vs baseline: 1.7244x; 1.3131x over previous
import functools

import jax
import jax.numpy as jnp
from jax import lax
from jax.experimental import pallas as pl
from jax.experimental.pallas import tpu as pltpu
from jax.experimental.pallas import tpu_sc as plsc

N = 4096
D = 256
E = 65536
DH = D // 2

def _dotT(x, w):
    return lax.dot_general(x, w, (((1,), (1,)), ((), ())),
                           preferred_element_type=jnp.float32)


_NC = 2
_NS = 16
_NW = _NC * _NS
_EPW = E // _NW


_RPW = N // _NW
_CAP = 3072
_CHB = 1024
_GB = 64
_GBS = 6
_SLAB = 8
_HALF = 0.5 * (N - 1)

_SC_PARAMS = dict(
    compiler_params=pltpu.CompilerParams(needs_layout_passes=False))


def _iota16():
    return lax.iota(jnp.int32, 16)


def _splat(x, dtype=jnp.int32):
    return jax.lax.broadcast(x, (16,)).astype(dtype)


def _sigm(x):
    return 1.0 / (1.0 + jnp.exp(-x))


def _mesh():
    return plsc.VectorSubcoreMesh(core_axis_name="c", subcore_axis_name="s")


def _sc_gather(s_tab, og_tab, r0, r1):
    gch = 64
    nchunk = _EPW // gch

    @functools.partial(
        pl.kernel, mesh=_mesh(), **_SC_PARAMS,
        out_type=[jax.ShapeDtypeStruct((E, D), jnp.float32),
                  jax.ShapeDtypeStruct((E, D), jnp.float32)],
        scratch_types=[pltpu.VMEM((_EPW,), jnp.int32),
                       pltpu.VMEM((_EPW,), jnp.int32),
                       pltpu.VMEM((2, gch, D), jnp.float32),
                       pltpu.VMEM((2, gch, D), jnp.float32),
                       pltpu.SemaphoreType.DMA((2,)),
                       pltpu.SemaphoreType.DMA((2,)),
                       pltpu.SemaphoreType.DMA((2,)),
                       pltpu.SemaphoreType.DMA((2,))],
    )
    def k(s_hbm, og_hbm, r0_hbm, r1_hbm, sg_hbm, ogg_hbm,
          i0, i1, sbuf, obuf, gs, go, os_, oo):
        wid = lax.axis_index("s") * _NC + lax.axis_index("c")
        base = wid * _EPW
        pltpu.sync_copy(r0_hbm.at[pl.ds(base, _EPW)], i0)
        pltpu.sync_copy(r1_hbm.at[pl.ds(base, _EPW)], i1)

        def g_cp(c, p):
            off = c * gch
            return (
                pltpu.make_async_copy(s_hbm.at[i0.at[pl.ds(off, gch)]],
                                      sbuf.at[p], gs.at[p]),
                pltpu.make_async_copy(og_hbm.at[i1.at[pl.ds(off, gch)]],
                                      obuf.at[p], go.at[p]))

        def o_cp(c, p):
            off = base + c * gch
            return (
                pltpu.make_async_copy(sbuf.at[p],
                                      sg_hbm.at[pl.ds(off, gch)], os_.at[p]),
                pltpu.make_async_copy(obuf.at[p],
                                      ogg_hbm.at[pl.ds(off, gch)], oo.at[p]))

        for cp in g_cp(0, 0):
            cp.start()
        for c in range(nchunk):
            p = c & 1
            if c + 1 < nchunk:
                if c >= 1:
                    for cp in o_cp(c - 1, 1 - p):
                        cp.wait()
                for cp in g_cp(c + 1, 1 - p):
                    cp.start()
            for cp in g_cp(c, p):
                cp.wait()
            for cp in o_cp(c, p):
                cp.start()
        for cp in o_cp(nchunk - 2, (nchunk - 2) & 1):
            cp.wait()
        for cp in o_cp(nchunk - 1, (nchunk - 1) & 1):
            cp.wait()

    return k(s_tab, og_tab, r0, r1)


def _sc_edges_a(key, coeff, t3, zf32, zi32, mi32):
    nchunk = E // _CHB

    @functools.partial(
        pl.kernel, mesh=_mesh(), **_SC_PARAMS,
        out_type=[jax.ShapeDtypeStruct((N * DH,), jnp.float32),
                  jax.ShapeDtypeStruct((N,), jnp.float32),
                  jax.ShapeDtypeStruct((_NW * _CAP,), jnp.int32),
                  jax.ShapeDtypeStruct((_NW * _CAP,), jnp.float32),
                  jax.ShapeDtypeStruct((_NW * 16,), jnp.int32)],
        scratch_types=[pltpu.VMEM((16,), jnp.int32),
                       pltpu.VMEM((2, _CHB), jnp.int32),
                       pltpu.VMEM((2, _CHB), jnp.float32),
                       pltpu.VMEM((_CAP,), jnp.int32),
                       pltpu.VMEM((_CAP,), jnp.int32),
                       pltpu.VMEM((_CAP,), jnp.float32),
                       pltpu.VMEM((_CAP,), jnp.int32),
                       pltpu.VMEM((_CAP,), jnp.float32),
                       pltpu.VMEM((_SLAB * N,), jnp.float32),
                       pltpu.VMEM((_SLAB * N,), jnp.int32),
                       pltpu.VMEM((_RPW,), jnp.float32),
                       pltpu.VMEM((2, _GB, DH), jnp.float32),
                       pltpu.VMEM((_RPW * DH,), jnp.float32),
                       pltpu.SemaphoreType.DMA((2,)),
                       pltpu.SemaphoreType.DMA((2,)),
                       pltpu.SemaphoreType.DMA((2,))],
    )
    def k(key_hbm, cf_hbm, t3_hbm, zf_hbm, zi_hbm, mi_hbm,
          c1_hbm, rs_hbm, eba_hbm, ed_hbm, cnt_hbm,
          cnt16, kb, cb, sa, sb, sc, se, sd, slab_c, slab_e, rowd,
          gbuf, c1acc, ksem, csem, gsem):
        wid = lax.axis_index("s") * _NC + lax.axis_index("c")
        lo = wid * _RPW
        klo = lo * N
        khi = (lo + _RPW) * N
        iota = _iota16()

        def ch_cp(c, p):
            return (
                pltpu.make_async_copy(key_hbm.at[pl.ds(c * _CHB, _CHB)],
                                      kb.at[p], ksem.at[p]),
                pltpu.make_async_copy(cf_hbm.at[pl.ds(c * _CHB, _CHB)],
                                      cb.at[p], csem.at[p]))

        for cp in ch_cp(0, 0):
            cp.start()

        pltpu.sync_copy(zf_hbm, slab_c)
        pltpu.sync_copy(zf_hbm.at[pl.ds(0, _RPW * DH)], c1acc)
        pltpu.sync_copy(zf_hbm.at[pl.ds(0, _CAP)], sd)
        pltpu.sync_copy(zf_hbm.at[pl.ds(0, _RPW)], rowd)
        pltpu.sync_copy(zi_hbm, sa)
        pltpu.sync_copy(zi_hbm, sb)
        pltpu.sync_copy(mi_hbm, se)

        cntv = jnp.zeros((16,), jnp.int32)
        for c in range(nchunk):
            p = c & 1
            if c + 1 < nchunk:
                for cp in ch_cp(c + 1, 1 - p):
                    cp.start()
            for cp in ch_cp(c, p):
                cp.wait()

            def scan_batch(i, cntv, _c=c, _p=p):
                kv = kb[_p, pl.ds(i * 16, 16)]
                m = (kv >= klo) & (kv < khi)
                pos = cntv + jnp.cumsum(m.astype(jnp.int32)) - 1
                al = (kv >> 12) - lo
                b = kv & (N - 1)
                cf = cb[_p, pl.ds(i * 16, 16)]
                eid = _splat(_c * _CHB + i * 16) + iota
                plsc.store_scatter(sa, [pos], al, mask=m)
                plsc.store_scatter(sb, [pos], b, mask=m)
                plsc.store_scatter(sc, [pos], cf, mask=m)
                plsc.store_scatter(se, [pos], eid, mask=m)
                return cntv + plsc.all_reduce_population_count(m)
            cntv = lax.fori_loop(0, _CHB // 16, scan_batch, cntv)
        count = jnp.sum(cntv) >> 4
        nbat = (count + 15) >> 4

        for sub in range(_RPW // _SLAB):
            def p1(i, _):
                al = sa[pl.ds(i * 16, 16)]
                e = se[pl.ds(i * 16, 16)]
                m = (e >= 0) & ((al >> 3) == sub)
                lidx = (al & (_SLAB - 1)) * N + sb[pl.ds(i * 16, 16)]
                plsc.addupdate_scatter(slab_c, [lidx],
                                       sc[pl.ds(i * 16, 16)], mask=m)
                plsc.store_scatter(slab_e, [lidx], e, mask=m)
                return 0
            lax.fori_loop(0, nbat, p1, 0)

            def p2(i, _):
                al = sa[pl.ds(i * 16, 16)]
                b = sb[pl.ds(i * 16, 16)]
                e = se[pl.ds(i * 16, 16)]
                m = (e >= 0) & ((al >> 3) == sub)
                lidx = (al & (_SLAB - 1)) * N + b
                cs = plsc.load_gather(slab_c, [lidx], mask=m)
                ce = plsc.load_gather(slab_e, [lidx], mask=m)
                rep = m & (ce == e) & ((al + lo) != b)
                delta = jnp.where(rep, _sigm(cs) - 0.5, 0.0)
                sd[pl.ds(i * 16, 16)] = sd[pl.ds(i * 16, 16)] + delta
                plsc.addupdate_scatter(rowd, [al], delta, mask=rep)
                plsc.store_scatter(slab_c, [lidx],
                                   jnp.zeros((16,), jnp.float32), mask=m)
                return 0
            lax.fori_loop(0, nbat, p2, 0)

        nch = (count + _GB - 1) >> _GBS

        def g_start(cs_, p):
            pltpu.make_async_copy(
                t3_hbm.at[sb.at[pl.ds(cs_ * _GB, _GB)]],
                gbuf.at[p], gsem.at[p]).start()

        def g_wait(cs_, p):
            pltpu.make_async_copy(
                t3_hbm.at[sb.at[pl.ds(cs_ * _GB, _GB)]],
                gbuf.at[p], gsem.at[p]).wait()

        def proc(cs_, p):
            def acc_edge(j, _):
                i = cs_ * _GB + j
                dsp = plsc.load_gather(sd, [_splat(i)])
                asp = plsc.load_gather(sa, [_splat(i)])
                for kk in range(DH // 16):
                    v = gbuf[p, j, pl.ds(kk * 16, 16)] * dsp
                    plsc.addupdate_scatter(c1acc,
                                           [asp * DH + kk * 16 + iota], v)
                return 0
            lax.fori_loop(0, _GB, acc_edge, 0)

        @pl.when(nch > 0)
        def _():
            g_start(0, 0)

        def pair(i, _):
            c0 = 2 * i

            @pl.when(c0 + 1 < nch)
            def _():
                g_start(c0 + 1, 1)
            g_wait(c0, 0)
            proc(c0, 0)

            @pl.when(c0 + 2 < nch)
            def _():
                g_start(c0 + 2, 0)

            @pl.when(c0 + 1 < nch)
            def _():
                g_wait(c0 + 1, 1)
                proc(c0 + 1, 1)
            return 0
        lax.fori_loop(0, (nch + 1) >> 1, pair, 0)

        def fin_batch(i, _):
            al = sa[pl.ds(i * 16, 16)]
            e = se[pl.ds(i * 16, 16)]
            valid = e >= 0
            rd = plsc.load_gather(rowd, [al], mask=valid)
            d2 = sd[pl.ds(i * 16, 16)] / (rd + _HALF)
            sd[pl.ds(i * 16, 16)] = jnp.where(valid, d2, 0.0)
            pb = (sb[pl.ds(i * 16, 16)] << 12) | (al + lo)
            sa[pl.ds(i * 16, 16)] = jnp.where(valid, pb, -1)
            return 0
        lax.fori_loop(0, _CAP // 16, fin_batch, 0)

        def rs_batch(i, _):
            rowd[pl.ds(i * 16, 16)] = rowd[pl.ds(i * 16, 16)] + _HALF
            return 0
        lax.fori_loop(0, _RPW // 16, rs_batch, 0)

        cnt16[pl.ds(0, 16)] = cntv
        pltpu.sync_copy(c1acc, c1_hbm.at[pl.ds(lo * DH, _RPW * DH)])
        pltpu.sync_copy(rowd, rs_hbm.at[pl.ds(lo, _RPW)])
        pltpu.sync_copy(sa, eba_hbm.at[pl.ds(wid * _CAP, _CAP)])
        pltpu.sync_copy(sd, ed_hbm.at[pl.ds(wid * _CAP, _CAP)])
        pltpu.sync_copy(cnt16, cnt_hbm.at[pl.ds(wid * 16, 16)])

    return k(key, coeff, t3, zf32, zi32, mi32)


def _sc_edges_b(eba, ed2, counts, t3, zf32, zi32):

    @functools.partial(
        pl.kernel, mesh=_mesh(), **_SC_PARAMS,
        out_type=jax.ShapeDtypeStruct((N * DH,), jnp.float32),
        scratch_types=[pltpu.VMEM((_NW * 16,), jnp.int32),
                       pltpu.VMEM((2, _CAP), jnp.int32),
                       pltpu.VMEM((2, _CAP), jnp.float32),
                       pltpu.VMEM((_CAP,), jnp.int32),
                       pltpu.VMEM((_CAP,), jnp.int32),
                       pltpu.VMEM((_CAP,), jnp.float32),
                       pltpu.VMEM((2, _GB, DH), jnp.float32),
                       pltpu.VMEM((_RPW * DH,), jnp.float32),
                       pltpu.SemaphoreType.DMA((2,)),
                       pltpu.SemaphoreType.DMA((2,)),
                       pltpu.SemaphoreType.DMA((2,))],
    )
    def k(eba_hbm, ed_hbm, cn_hbm, t3_hbm, zf_hbm, zi_hbm, c2_hbm,
          cnb, kb, db, sga, sbl, sd2, gbuf, c2acc, ksem, dsem, gsem):
        wid = lax.axis_index("s") * _NC + lax.axis_index("c")
        lo = wid * _RPW
        klo = lo << 12
        khi = (lo + _RPW) << 12
        iota = _iota16()

        def reg_cp(s, p):
            return (
                pltpu.make_async_copy(eba_hbm.at[pl.ds(s * _CAP, _CAP)],
                                      kb.at[p], ksem.at[p]),
                pltpu.make_async_copy(ed_hbm.at[pl.ds(s * _CAP, _CAP)],
                                      db.at[p], dsem.at[p]))

        for cp in reg_cp(0, 0):
            cp.start()
        pltpu.sync_copy(cn_hbm, cnb)
        pltpu.sync_copy(zf_hbm.at[pl.ds(0, _RPW * DH)], c2acc)
        pltpu.sync_copy(zf_hbm.at[pl.ds(0, _CAP)], sd2)
        pltpu.sync_copy(zi_hbm, sga)
        pltpu.sync_copy(zi_hbm, sbl)

        cntv = jnp.zeros((16,), jnp.int32)
        for s in range(_NW):
            p = s & 1
            if s + 1 < _NW:
                for cp in reg_cp(s + 1, 1 - p):
                    cp.start()
            for cp in reg_cp(s, p):
                cp.wait()
            scnt = jnp.sum(cnb[pl.ds(s * 16, 16)]) >> 4
            nb = (scnt + 15) >> 4

            def scan_batch(i, cntv, _p=p):
                pb = kb[_p, pl.ds(i * 16, 16)]
                m = (pb >= klo) & (pb < khi)
                pos = cntv + jnp.cumsum(m.astype(jnp.int32)) - 1
                bl = (pb >> 12) - lo
                a = pb & (N - 1)
                d2 = db[_p, pl.ds(i * 16, 16)]
                plsc.store_scatter(sga, [pos], a, mask=m)
                plsc.store_scatter(sbl, [pos], bl, mask=m)
                plsc.store_scatter(sd2, [pos], d2, mask=m)
                return cntv + plsc.all_reduce_population_count(m)
            cntv = lax.fori_loop(0, nb, scan_batch, cntv)
        count = jnp.sum(cntv) >> 4
        nch = (count + _GB - 1) >> _GBS

        def g_start(cs_, p):
            pltpu.make_async_copy(
                t3_hbm.at[sga.at[pl.ds(cs_ * _GB, _GB)]],
                gbuf.at[p], gsem.at[p]).start()

        def g_wait(cs_, p):
            pltpu.make_async_copy(
                t3_hbm.at[sga.at[pl.ds(cs_ * _GB, _GB)]],
                gbuf.at[p], gsem.at[p]).wait()

        def proc(cs_, p):
            def acc_edge(j, _):
                i = cs_ * _GB + j
                dsp = plsc.load_gather(sd2, [_splat(i)])
                bsp = plsc.load_gather(sbl, [_splat(i)])
                for kk in range(DH // 16):
                    v = gbuf[p, j, pl.ds(kk * 16, 16)] * dsp
                    plsc.addupdate_scatter(c2acc,
                                           [bsp * DH + kk * 16 + iota], v)
                return 0
            lax.fori_loop(0, _GB, acc_edge, 0)

        @pl.when(nch > 0)
        def _():
            g_start(0, 0)

        def pair(i, _):
            c0 = 2 * i

            @pl.when(c0 + 1 < nch)
            def _():
                g_start(c0 + 1, 1)
            g_wait(c0, 0)
            proc(c0, 0)

            @pl.when(c0 + 2 < nch)
            def _():
                g_start(c0 + 2, 0)

            @pl.when(c0 + 1 < nch)
            def _():
                g_wait(c0 + 1, 1)
                proc(c0 + 1, 1)
            return 0
        lax.fori_loop(0, (nch + 1) >> 1, pair, 0)

        pltpu.sync_copy(c2acc, c2_hbm.at[pl.ds(lo * DH, _RPW * DH)])

    return k(eba, ed2, counts, t3, zf32, zi32)


def _prep_body(obj_ref, ws_ref, wsb_ref, wo_ref, wob_ref, ww_ref, wt3_ref,
               wt3b_ref, s_ref, og_ref, t3_ref):
    x = obj_ref[...]
    s_ref[...] = _dotT(x, ws_ref[...]) + wsb_ref[...]
    og_ref[...] = (_dotT(x, wo_ref[...]) + wob_ref[...]) * ww_ref[...]
    t3_ref[...] = jnp.maximum(_dotT(x, wt3_ref[...]) + wt3b_ref[...], 0.0)


def _prep(obj, ws_w, ws_b, wo_w, wo_b, w_w, wt3_w, wt3_b):
    blk = 512
    grid = N // blk
    full = lambda shape: pl.BlockSpec(shape, lambda i: (0, 0))
    return pl.pallas_call(
        _prep_body,
        grid=(grid,),
        in_specs=[
            pl.BlockSpec((blk, D), lambda i: (i, 0)),
            full((D, D)), full((1, D)),
            full((D, D)), full((1, D)),
            full((1, D)),
            full((DH, D)), full((1, DH)),
        ],
        out_specs=[
            pl.BlockSpec((blk, D), lambda i: (i, 0)),
            pl.BlockSpec((blk, D), lambda i: (i, 0)),
            pl.BlockSpec((blk, DH), lambda i: (i, 0)),
        ],
        out_shape=[
            jax.ShapeDtypeStruct((N, D), jnp.float32),
            jax.ShapeDtypeStruct((N, D), jnp.float32),
            jax.ShapeDtypeStruct((N, DH), jnp.float32),
        ],
    )(obj, ws_w, ws_b.reshape(1, D), wo_w, wo_b.reshape(1, D),
      w_w.reshape(1, D), wt3_w, wt3_b.reshape(1, DH))


def _coeff_body(union_ref, sg_ref, og_ref, wu_ref, wub_ref, wb_ref, out_ref):
    u = _dotT(union_ref[...], wu_ref[...]) + wub_ref[...]
    p = sg_ref[...] * og_ref[...] * u
    rows = out_ref.shape[0]
    out_ref[...] = (jnp.sum(p, axis=1) + wb_ref[0, 0]).reshape(rows, 256)


def _coeff(union, sg, og, wu_w, wu_b, w_b):
    blk = 4096
    grid = E // blk
    rows = blk // 256
    full = lambda shape: pl.BlockSpec(shape, lambda i: (0, 0))
    out = pl.pallas_call(
        _coeff_body,
        grid=(grid,),
        in_specs=[
            pl.BlockSpec((blk, D), lambda i: (i, 0)),
            pl.BlockSpec((blk, D), lambda i: (i, 0)),
            pl.BlockSpec((blk, D), lambda i: (i, 0)),
            full((D, D)), full((1, D)), full((1, D)),
        ],
        out_specs=pl.BlockSpec((rows, 256), lambda i: (i, 0)),
        out_shape=jax.ShapeDtypeStruct((E // 256, 256), jnp.float32),
    )(union, sg, og, wu_w, wu_b.reshape(1, D),
      jnp.broadcast_to(w_b.reshape(1, 1), (1, D)))
    return out.reshape(E)


def _mlp_body(obj_ref, t3full_ref, rsfull_ref, c1_ref, c2_ref, t1_ref,
              t1b_ref, lng_ref, lnb_ref, t2_ref, t2b_ref, out_ref):
    i = pl.program_id(0)
    blk = obj_ref.shape[0]
    t3full = t3full_ref[...]
    rsfull = rsfull_ref[...]
    csum1 = jnp.sum(t3full, axis=0, keepdims=True)
    csum2 = jnp.sum(t3full / rsfull, axis=0, keepdims=True)
    t3_blk = t3full_ref[pl.ds(i * blk, blk), :]
    rs_blk = rsfull_ref[pl.ds(i * blk, blk), :]
    ctx1 = (0.5 * csum1 - 0.5 * t3_blk + c1_ref[...]) / rs_blk
    ctx2 = 0.5 * csum2 - 0.5 * t3_blk / rs_blk + c2_ref[...]
    ctx = jnp.concatenate([ctx1, ctx2], axis=1)
    h = _dotT(ctx, t1_ref[...]) + t1b_ref[...]
    mu = jnp.mean(h, axis=1, keepdims=True)
    dh = h - mu
    var = jnp.mean(dh * dh, axis=1, keepdims=True)
    h = dh * lax.rsqrt(var + 1e-5) * lng_ref[...] + lnb_ref[...]
    h = jnp.maximum(h, 0.0)
    nb = _dotT(h, t2_ref[...]) + t2b_ref[...]
    out_ref[...] = jnp.maximum(obj_ref[...] + nb, 0.0)


def _mlp(obj, t3, rowsum, c1, c2, t1_w, t1_b, ln_g, ln_b, t2_w, t2_b):
    blk = 512
    grid = N // blk
    full = lambda shape: pl.BlockSpec(shape, lambda i: (0, 0))
    Dq = D // 4
    return pl.pallas_call(
        _mlp_body,
        grid=(grid,),
        in_specs=[
            pl.BlockSpec((blk, D), lambda i: (i, 0)),
            full((N, DH)), full((N, 1)),
            pl.BlockSpec((blk, DH), lambda i: (i, 0)),
            pl.BlockSpec((blk, DH), lambda i: (i, 0)),
            full((Dq, D)), full((1, Dq)), full((1, Dq)), full((1, Dq)),
            full((D, Dq)), full((1, D)),
        ],
        out_specs=pl.BlockSpec((blk, D), lambda i: (i, 0)),
        out_shape=jax.ShapeDtypeStruct((N, D), jnp.float32),
    )(obj, t3, rowsum.reshape(N, 1), c1, c2, t1_w, t1_b.reshape(1, Dq),
      ln_g.reshape(1, Dq), ln_b.reshape(1, Dq), t2_w, t2_b.reshape(1, D))


def kernel(obj_feats, union_feats, rel_pair_idx, ws_w, ws_b, wo_w, wo_b,
           wu_w, wu_b, w_w, w_b, wt3_w, wt3_b, t1_w, t1_b, ln_g, ln_b,
           t2_w, t2_b):
    r0 = rel_pair_idx[:, 0].astype(jnp.int32)
    r1 = rel_pair_idx[:, 1].astype(jnp.int32)

    s_tab, og_tab, t3 = _prep(obj_feats, ws_w, ws_b, wo_w, wo_b, w_w,
                              wt3_w, wt3_b)

    sg, og = _sc_gather(s_tab, og_tab, r0, r1)

    coeff = _coeff(union_feats, sg, og, wu_w, wu_b, w_b)

    key = r0 * N + r1
    zf32 = jnp.zeros((_SLAB * N,), jnp.float32)
    zi32 = jnp.zeros((_CAP,), jnp.int32)
    mi32 = jnp.full((_CAP,), -1, jnp.int32)
    c1f, rowsum, eba, ed2, cnts = _sc_edges_a(key, coeff, t3, zf32, zi32,
                                              mi32)
    c2f = _sc_edges_b(eba, ed2, cnts, t3, zf32, zi32)
    c1 = c1f.reshape(N, DH)
    c2 = c2f.reshape(N, DH)

    return _mlp(obj_feats, t3, rowsum, c1, c2, t1_w, t1_b, ln_g, ln_b,
                t2_w, t2_b)


# speedup vs baseline: 1.7302x; 1.0034x over previous
import functools

import jax
import jax.numpy as jnp
from jax import lax
from jax.experimental import pallas as pl
from jax.experimental.pallas import tpu as pltpu
from jax.experimental.pallas import tpu_sc as plsc

N = 4096
D = 256
E = 65536
DH = D // 2

def _dotT(x, w):
    return lax.dot_general(x, w, (((1,), (1,)), ((), ())),
                           preferred_element_type=jnp.float32)


_NC = 2
_NS = 16
_NW = _NC * _NS
_EPW = E // _NW


_RPW = N // _NW
_CAP = 3136
_CAPP = _CAP + 16
_CHB = 1024
_GB = 64
_GBS = 6
_SLAB = 8
_HALF = 0.5 * (N - 1)

_SC_PARAMS = dict(
    compiler_params=pltpu.CompilerParams(needs_layout_passes=False))


def _iota16():
    return lax.iota(jnp.int32, 16)


def _splat(x, dtype=jnp.int32):
    return jax.lax.broadcast(x, (16,)).astype(dtype)


def _sigm(x):
    return 1.0 / (1.0 + jnp.exp(-x))


def _mesh():
    return plsc.VectorSubcoreMesh(core_axis_name="c", subcore_axis_name="s")


def _sc_gather(s_tab, og_tab, r0, r1):
    gch = 64
    nchunk = _EPW // gch

    @functools.partial(
        pl.kernel, mesh=_mesh(), **_SC_PARAMS,
        out_type=[jax.ShapeDtypeStruct((E, D), jnp.float32),
                  jax.ShapeDtypeStruct((E, D), jnp.float32)],
        scratch_types=[pltpu.VMEM((_EPW,), jnp.int32),
                       pltpu.VMEM((_EPW,), jnp.int32),
                       pltpu.VMEM((2, gch, D), jnp.float32),
                       pltpu.VMEM((2, gch, D), jnp.float32),
                       pltpu.SemaphoreType.DMA((2,)),
                       pltpu.SemaphoreType.DMA((2,)),
                       pltpu.SemaphoreType.DMA((2,)),
                       pltpu.SemaphoreType.DMA((2,))],
    )
    def k(s_hbm, og_hbm, r0_hbm, r1_hbm, sg_hbm, ogg_hbm,
          i0, i1, sbuf, obuf, gs, go, os_, oo):
        wid = lax.axis_index("s") * _NC + lax.axis_index("c")
        base = wid * _EPW
        pltpu.sync_copy(r0_hbm.at[pl.ds(base, _EPW)], i0)
        pltpu.sync_copy(r1_hbm.at[pl.ds(base, _EPW)], i1)

        def g_cp(c, p):
            off = c * gch
            return (
                pltpu.make_async_copy(s_hbm.at[i0.at[pl.ds(off, gch)]],
                                      sbuf.at[p], gs.at[p]),
                pltpu.make_async_copy(og_hbm.at[i1.at[pl.ds(off, gch)]],
                                      obuf.at[p], go.at[p]))

        def o_cp(c, p):
            off = base + c * gch
            return (
                pltpu.make_async_copy(sbuf.at[p],
                                      sg_hbm.at[pl.ds(off, gch)], os_.at[p]),
                pltpu.make_async_copy(obuf.at[p],
                                      ogg_hbm.at[pl.ds(off, gch)], oo.at[p]))

        for cp in g_cp(0, 0):
            cp.start()
        for c in range(nchunk):
            p = c & 1
            if c + 1 < nchunk:
                if c >= 1:
                    for cp in o_cp(c - 1, 1 - p):
                        cp.wait()
                for cp in g_cp(c + 1, 1 - p):
                    cp.start()
            for cp in g_cp(c, p):
                cp.wait()
            for cp in o_cp(c, p):
                cp.start()
        for cp in o_cp(nchunk - 2, (nchunk - 2) & 1):
            cp.wait()
        for cp in o_cp(nchunk - 1, (nchunk - 1) & 1):
            cp.wait()

    return k(s_tab, og_tab, r0, r1)


def _sc_edges_a(key, coeff, t3, zf32, zi32, mi32):
    nchunk = E // _CHB

    @functools.partial(
        pl.kernel, mesh=_mesh(), **_SC_PARAMS,
        out_type=[jax.ShapeDtypeStruct((N * DH,), jnp.float32),
                  jax.ShapeDtypeStruct((N,), jnp.float32),
                  jax.ShapeDtypeStruct((_NW * _CAP,), jnp.int32),
                  jax.ShapeDtypeStruct((_NW * _CAP,), jnp.float32),
                  jax.ShapeDtypeStruct((_NW * 16,), jnp.int32)],
        scratch_types=[pltpu.VMEM((16,), jnp.int32),
                       pltpu.VMEM((2, _CHB), jnp.int32),
                       pltpu.VMEM((2, _CHB), jnp.float32),
                       pltpu.VMEM((_CAPP,), jnp.int32),
                       pltpu.VMEM((_CAPP,), jnp.int32),
                       pltpu.VMEM((_CAPP,), jnp.float32),
                       pltpu.VMEM((_CAPP,), jnp.int32),
                       pltpu.VMEM((_CAPP,), jnp.float32),
                       pltpu.VMEM((_SLAB * N,), jnp.float32),
                       pltpu.VMEM((_SLAB * N,), jnp.int32),
                       pltpu.VMEM((_RPW,), jnp.float32),
                       pltpu.VMEM((2, _GB, DH), jnp.float32),
                       pltpu.VMEM((_RPW * DH,), jnp.float32),
                       pltpu.SemaphoreType.DMA((2,)),
                       pltpu.SemaphoreType.DMA((2,)),
                       pltpu.SemaphoreType.DMA((2,))],
    )
    def k(key_hbm, cf_hbm, t3_hbm, zf_hbm, zi_hbm, mi_hbm,
          c1_hbm, rs_hbm, eba_hbm, ed_hbm, cnt_hbm,
          cnt16, kb, cb, sa, sb, sc, se, sd, slab_c, slab_e, rowd,
          gbuf, c1acc, ksem, csem, gsem):
        wid = lax.axis_index("s") * _NC + lax.axis_index("c")
        lo = wid * _RPW
        klo = lo * N
        khi = (lo + _RPW) * N
        iota = _iota16()

        def ch_cp(c, p):
            return (
                pltpu.make_async_copy(key_hbm.at[pl.ds(c * _CHB, _CHB)],
                                      kb.at[p], ksem.at[p]),
                pltpu.make_async_copy(cf_hbm.at[pl.ds(c * _CHB, _CHB)],
                                      cb.at[p], csem.at[p]))

        for cp in ch_cp(0, 0):
            cp.start()

        pltpu.sync_copy(zf_hbm, slab_c)
        pltpu.sync_copy(zf_hbm.at[pl.ds(0, _RPW * DH)], c1acc)
        pltpu.sync_copy(zf_hbm.at[pl.ds(0, _CAPP)], sd)
        pltpu.sync_copy(zf_hbm.at[pl.ds(0, _RPW)], rowd)
        pltpu.sync_copy(zi_hbm, sa)
        pltpu.sync_copy(zi_hbm, sb)
        pltpu.sync_copy(zf_hbm.at[pl.ds(0, _CAPP)], sc)
        pltpu.sync_copy(mi_hbm, se)

        cntv = jnp.zeros((16,), jnp.int32)
        for c in range(nchunk):
            p = c & 1
            if c + 1 < nchunk:
                for cp in ch_cp(c + 1, 1 - p):
                    cp.start()
            for cp in ch_cp(c, p):
                cp.wait()

            def scan_pair(ii, cntv, _c=c, _p=p):
                for u in range(2):
                    i = ii * 2 + u
                    kv = kb[_p, pl.ds(i * 16, 16)]
                    m = (kv >= klo) & (kv < khi)
                    pos = cntv + jnp.cumsum(m.astype(jnp.int32)) - 1
                    al = (kv >> 12) - lo
                    b = kv & (N - 1)
                    cf = cb[_p, pl.ds(i * 16, 16)]
                    eid = _splat(_c * _CHB + i * 16) + iota
                    plsc.store_scatter(sa, [pos], al, mask=m)
                    plsc.store_scatter(sb, [pos], b, mask=m)
                    plsc.store_scatter(sc, [pos], cf, mask=m)
                    plsc.store_scatter(se, [pos], eid, mask=m)
                    cntv = cntv + plsc.all_reduce_population_count(m)
                return cntv
            cntv = lax.fori_loop(0, _CHB // 32, scan_pair, cntv)
        count = jnp.sum(cntv) >> 4
        nbat = (count + 15) >> 4

        nbat2 = (nbat + 1) >> 1
        for sub in range(_RPW // _SLAB):
            def p1(ii, _):
                for u in range(2):
                    i = ii * 2 + u
                    al = sa[pl.ds(i * 16, 16)]
                    e = se[pl.ds(i * 16, 16)]
                    m = (e >= 0) & ((al >> 3) == sub)
                    lidx = (al & (_SLAB - 1)) * N + sb[pl.ds(i * 16, 16)]
                    plsc.addupdate_scatter(slab_c, [lidx],
                                           sc[pl.ds(i * 16, 16)], mask=m)
                    plsc.store_scatter(slab_e, [lidx], e, mask=m)
                return 0
            lax.fori_loop(0, nbat2, p1, 0)

            def p2(ii, _):
                for u in range(2):
                    i = ii * 2 + u
                    al = sa[pl.ds(i * 16, 16)]
                    b = sb[pl.ds(i * 16, 16)]
                    e = se[pl.ds(i * 16, 16)]
                    m = (e >= 0) & ((al >> 3) == sub)
                    lidx = (al & (_SLAB - 1)) * N + b
                    cs = plsc.load_gather(slab_c, [lidx], mask=m)
                    ce = plsc.load_gather(slab_e, [lidx], mask=m)
                    rep = m & (ce == e) & ((al + lo) != b)
                    delta = jnp.where(rep, _sigm(cs) - 0.5, 0.0)
                    sd[pl.ds(i * 16, 16)] = sd[pl.ds(i * 16, 16)] + delta
                    plsc.addupdate_scatter(rowd, [al], delta, mask=rep)
                    plsc.store_scatter(slab_c, [lidx],
                                       jnp.zeros((16,), jnp.float32),
                                       mask=m)
                return 0
            lax.fori_loop(0, nbat2, p2, 0)

        nch = (count + _GB - 1) >> _GBS

        def g_start(cs_, p):
            pltpu.make_async_copy(
                t3_hbm.at[sb.at[pl.ds(cs_ * _GB, _GB)]],
                gbuf.at[p], gsem.at[p]).start()

        def g_wait(cs_, p):
            pltpu.make_async_copy(
                t3_hbm.at[sb.at[pl.ds(cs_ * _GB, _GB)]],
                gbuf.at[p], gsem.at[p]).wait()

        def proc(cs_, p):
            def acc_edge(jj, _):
                for u in range(4):
                    j = jj * 4 + u
                    i = cs_ * _GB + j
                    dsp = plsc.load_gather(sd, [_splat(i)])
                    asp = plsc.load_gather(sa, [_splat(i)])
                    base = asp * DH + iota
                    for kk in range(DH // 16):
                        v = gbuf[p, j, pl.ds(kk * 16, 16)] * dsp
                        plsc.addupdate_scatter(c1acc, [base + kk * 16], v)
                return 0
            lax.fori_loop(0, _GB // 4, acc_edge, 0)

        @pl.when(nch > 0)
        def _():
            g_start(0, 0)

        def pair(i, _):
            c0 = 2 * i

            @pl.when(c0 + 1 < nch)
            def _():
                g_start(c0 + 1, 1)
            g_wait(c0, 0)
            proc(c0, 0)

            @pl.when(c0 + 2 < nch)
            def _():
                g_start(c0 + 2, 0)

            @pl.when(c0 + 1 < nch)
            def _():
                g_wait(c0 + 1, 1)
                proc(c0 + 1, 1)
            return 0
        lax.fori_loop(0, (nch + 1) >> 1, pair, 0)

        def fin_batch(i, _):
            al = sa[pl.ds(i * 16, 16)]
            e = se[pl.ds(i * 16, 16)]
            valid = e >= 0
            rd = plsc.load_gather(rowd, [al], mask=valid)
            d2 = sd[pl.ds(i * 16, 16)] / (rd + _HALF)
            sd[pl.ds(i * 16, 16)] = jnp.where(valid, d2, 0.0)
            pb = (sb[pl.ds(i * 16, 16)] << 12) | (al + lo)
            sa[pl.ds(i * 16, 16)] = jnp.where(valid, pb, -1)
            return 0
        lax.fori_loop(0, _CAP // 16, fin_batch, 0)

        def rs_batch(i, _):
            rowd[pl.ds(i * 16, 16)] = rowd[pl.ds(i * 16, 16)] + _HALF
            return 0
        lax.fori_loop(0, _RPW // 16, rs_batch, 0)

        cnt16[pl.ds(0, 16)] = cntv
        pltpu.sync_copy(c1acc, c1_hbm.at[pl.ds(lo * DH, _RPW * DH)])
        pltpu.sync_copy(rowd, rs_hbm.at[pl.ds(lo, _RPW)])
        pltpu.sync_copy(sa.at[pl.ds(0, _CAP)],
                        eba_hbm.at[pl.ds(wid * _CAP, _CAP)])
        pltpu.sync_copy(sd.at[pl.ds(0, _CAP)],
                        ed_hbm.at[pl.ds(wid * _CAP, _CAP)])
        pltpu.sync_copy(cnt16, cnt_hbm.at[pl.ds(wid * 16, 16)])

    return k(key, coeff, t3, zf32, zi32, mi32)


def _sc_edges_b(eba, ed2, counts, t3, zf32, zi32):

    @functools.partial(
        pl.kernel, mesh=_mesh(), **_SC_PARAMS,
        out_type=jax.ShapeDtypeStruct((N * DH,), jnp.float32),
        scratch_types=[pltpu.VMEM((_NW * 16,), jnp.int32),
                       pltpu.VMEM((_CAPP,), jnp.int32),
                       pltpu.VMEM((_CAPP,), jnp.int32),
                       pltpu.VMEM((_CAPP,), jnp.float32),
                       pltpu.VMEM((_CAPP,), jnp.float32),
                       pltpu.VMEM((_CAPP,), jnp.int32),
                       pltpu.VMEM((_CAPP,), jnp.int32),
                       pltpu.VMEM((_CAPP,), jnp.float32),
                       pltpu.VMEM((2, _GB, DH), jnp.float32),
                       pltpu.VMEM((_RPW * DH,), jnp.float32),
                       pltpu.SemaphoreType.DMA((2,)),
                       pltpu.SemaphoreType.DMA((2,)),
                       pltpu.SemaphoreType.DMA((2,))],
    )
    def k(eba_hbm, ed_hbm, cn_hbm, t3_hbm, zf_hbm, zi_hbm, c2_hbm,
          cnb, kb0, kb1, db0, db1, sga, sbl, sd2, gbuf, c2acc,
          ksem, dsem, gsem):
        kbs = (kb0, kb1)
        dbs = (db0, db1)
        wid = lax.axis_index("s") * _NC + lax.axis_index("c")
        lo = wid * _RPW
        klo = lo << 12
        khi = (lo + _RPW) << 12
        iota = _iota16()

        def reg_cp(s, p):
            return (
                pltpu.make_async_copy(eba_hbm.at[pl.ds(s * _CAP, _CAP)],
                                      kbs[p].at[pl.ds(0, _CAP)], ksem.at[p]),
                pltpu.make_async_copy(ed_hbm.at[pl.ds(s * _CAP, _CAP)],
                                      dbs[p].at[pl.ds(0, _CAP)], dsem.at[p]))

        for cp in reg_cp(0, 0):
            cp.start()
        pltpu.sync_copy(cn_hbm, cnb)
        pltpu.sync_copy(zf_hbm.at[pl.ds(0, _RPW * DH)], c2acc)
        pltpu.sync_copy(zf_hbm.at[pl.ds(0, _CAPP)], sd2)
        pltpu.sync_copy(zi_hbm, sga)
        pltpu.sync_copy(zi_hbm, sbl)
        kb0[pl.ds(_CAP, 16)] = jnp.full((16,), -1, jnp.int32)
        kb1[pl.ds(_CAP, 16)] = jnp.full((16,), -1, jnp.int32)

        cntv = jnp.zeros((16,), jnp.int32)
        for s in range(_NW):
            p = s & 1
            if s + 1 < _NW:
                for cp in reg_cp(s + 1, 1 - p):
                    cp.start()
            for cp in reg_cp(s, p):
                cp.wait()
            scnt = jnp.sum(cnb[pl.ds(s * 16, 16)]) >> 4
            nb2 = (scnt + 31) >> 5

            def scan_pair(ii, cntv, _kb=kbs[p], _db=dbs[p]):
                for u in range(2):
                    i = ii * 2 + u
                    pb = _kb[pl.ds(i * 16, 16)]
                    m = (pb >= klo) & (pb < khi)
                    pos = cntv + jnp.cumsum(m.astype(jnp.int32)) - 1
                    bl = (pb >> 12) - lo
                    a = pb & (N - 1)
                    d2 = _db[pl.ds(i * 16, 16)]
                    plsc.store_scatter(sga, [pos], a, mask=m)
                    plsc.store_scatter(sbl, [pos], bl, mask=m)
                    plsc.store_scatter(sd2, [pos], d2, mask=m)
                    cntv = cntv + plsc.all_reduce_population_count(m)
                return cntv
            cntv = lax.fori_loop(0, nb2, scan_pair, cntv)
        count = jnp.sum(cntv) >> 4
        nch = (count + _GB - 1) >> _GBS

        def g_start(cs_, p):
            pltpu.make_async_copy(
                t3_hbm.at[sga.at[pl.ds(cs_ * _GB, _GB)]],
                gbuf.at[p], gsem.at[p]).start()

        def g_wait(cs_, p):
            pltpu.make_async_copy(
                t3_hbm.at[sga.at[pl.ds(cs_ * _GB, _GB)]],
                gbuf.at[p], gsem.at[p]).wait()

        def proc(cs_, p):
            def acc_edge(jj, _):
                for u in range(4):
                    j = jj * 4 + u
                    i = cs_ * _GB + j
                    dsp = plsc.load_gather(sd2, [_splat(i)])
                    bsp = plsc.load_gather(sbl, [_splat(i)])
                    base = bsp * DH + iota
                    for kk in range(DH // 16):
                        v = gbuf[p, j, pl.ds(kk * 16, 16)] * dsp
                        plsc.addupdate_scatter(c2acc, [base + kk * 16], v)
                return 0
            lax.fori_loop(0, _GB // 4, acc_edge, 0)

        @pl.when(nch > 0)
        def _():
            g_start(0, 0)

        def pair(i, _):
            c0 = 2 * i

            @pl.when(c0 + 1 < nch)
            def _():
                g_start(c0 + 1, 1)
            g_wait(c0, 0)
            proc(c0, 0)

            @pl.when(c0 + 2 < nch)
            def _():
                g_start(c0 + 2, 0)

            @pl.when(c0 + 1 < nch)
            def _():
                g_wait(c0 + 1, 1)
                proc(c0 + 1, 1)
            return 0
        lax.fori_loop(0, (nch + 1) >> 1, pair, 0)

        pltpu.sync_copy(c2acc, c2_hbm.at[pl.ds(lo * DH, _RPW * DH)])

    return k(eba, ed2, counts, t3, zf32, zi32)


def _prep_body(obj_ref, ws_ref, wsb_ref, wo_ref, wob_ref, ww_ref, wt3_ref,
               wt3b_ref, s_ref, og_ref, t3_ref):
    x = obj_ref[...]
    s_ref[...] = _dotT(x, ws_ref[...]) + wsb_ref[...]
    og_ref[...] = (_dotT(x, wo_ref[...]) + wob_ref[...]) * ww_ref[...]
    t3_ref[...] = jnp.maximum(_dotT(x, wt3_ref[...]) + wt3b_ref[...], 0.0)


def _prep(obj, ws_w, ws_b, wo_w, wo_b, w_w, wt3_w, wt3_b):
    blk = 512
    grid = N // blk
    full = lambda shape: pl.BlockSpec(shape, lambda i: (0, 0))
    return pl.pallas_call(
        _prep_body,
        grid=(grid,),
        in_specs=[
            pl.BlockSpec((blk, D), lambda i: (i, 0)),
            full((D, D)), full((1, D)),
            full((D, D)), full((1, D)),
            full((1, D)),
            full((DH, D)), full((1, DH)),
        ],
        out_specs=[
            pl.BlockSpec((blk, D), lambda i: (i, 0)),
            pl.BlockSpec((blk, D), lambda i: (i, 0)),
            pl.BlockSpec((blk, DH), lambda i: (i, 0)),
        ],
        out_shape=[
            jax.ShapeDtypeStruct((N, D), jnp.float32),
            jax.ShapeDtypeStruct((N, D), jnp.float32),
            jax.ShapeDtypeStruct((N, DH), jnp.float32),
        ],
    )(obj, ws_w, ws_b.reshape(1, D), wo_w, wo_b.reshape(1, D),
      w_w.reshape(1, D), wt3_w, wt3_b.reshape(1, DH))


def _coeff_body(union_ref, sg_ref, og_ref, wu_ref, wub_ref, wb_ref, out_ref):
    u = _dotT(union_ref[...], wu_ref[...]) + wub_ref[...]
    p = sg_ref[...] * og_ref[...] * u
    rows = out_ref.shape[0]
    out_ref[...] = (jnp.sum(p, axis=1) + wb_ref[0, 0]).reshape(rows, 256)


def _coeff(union, sg, og, wu_w, wu_b, w_b):
    blk = 4096
    grid = E // blk
    rows = blk // 256
    full = lambda shape: pl.BlockSpec(shape, lambda i: (0, 0))
    out = pl.pallas_call(
        _coeff_body,
        grid=(grid,),
        in_specs=[
            pl.BlockSpec((blk, D), lambda i: (i, 0)),
            pl.BlockSpec((blk, D), lambda i: (i, 0)),
            pl.BlockSpec((blk, D), lambda i: (i, 0)),
            full((D, D)), full((1, D)), full((1, D)),
        ],
        out_specs=pl.BlockSpec((rows, 256), lambda i: (i, 0)),
        out_shape=jax.ShapeDtypeStruct((E // 256, 256), jnp.float32),
    )(union, sg, og, wu_w, wu_b.reshape(1, D),
      jnp.broadcast_to(w_b.reshape(1, 1), (1, D)))
    return out.reshape(E)


def _mlp_body(obj_ref, t3full_ref, rsfull_ref, c1_ref, c2_ref, t1_ref,
              t1b_ref, lng_ref, lnb_ref, t2_ref, t2b_ref, out_ref):
    i = pl.program_id(0)
    blk = obj_ref.shape[0]
    t3full = t3full_ref[...]
    rsfull = rsfull_ref[...]
    csum1 = jnp.sum(t3full, axis=0, keepdims=True)
    csum2 = jnp.sum(t3full / rsfull, axis=0, keepdims=True)
    t3_blk = t3full_ref[pl.ds(i * blk, blk), :]
    rs_blk = rsfull_ref[pl.ds(i * blk, blk), :]
    ctx1 = (0.5 * csum1 - 0.5 * t3_blk + c1_ref[...]) / rs_blk
    ctx2 = 0.5 * csum2 - 0.5 * t3_blk / rs_blk + c2_ref[...]
    ctx = jnp.concatenate([ctx1, ctx2], axis=1)
    h = _dotT(ctx, t1_ref[...]) + t1b_ref[...]
    mu = jnp.mean(h, axis=1, keepdims=True)
    dh = h - mu
    var = jnp.mean(dh * dh, axis=1, keepdims=True)
    h = dh * lax.rsqrt(var + 1e-5) * lng_ref[...] + lnb_ref[...]
    h = jnp.maximum(h, 0.0)
    nb = _dotT(h, t2_ref[...]) + t2b_ref[...]
    out_ref[...] = jnp.maximum(obj_ref[...] + nb, 0.0)


def _mlp(obj, t3, rowsum, c1, c2, t1_w, t1_b, ln_g, ln_b, t2_w, t2_b):
    blk = 512
    grid = N // blk
    full = lambda shape: pl.BlockSpec(shape, lambda i: (0, 0))
    Dq = D // 4
    return pl.pallas_call(
        _mlp_body,
        grid=(grid,),
        in_specs=[
            pl.BlockSpec((blk, D), lambda i: (i, 0)),
            full((N, DH)), full((N, 1)),
            pl.BlockSpec((blk, DH), lambda i: (i, 0)),
            pl.BlockSpec((blk, DH), lambda i: (i, 0)),
            full((Dq, D)), full((1, Dq)), full((1, Dq)), full((1, Dq)),
            full((D, Dq)), full((1, D)),
        ],
        out_specs=pl.BlockSpec((blk, D), lambda i: (i, 0)),
        out_shape=jax.ShapeDtypeStruct((N, D), jnp.float32),
    )(obj, t3, rowsum.reshape(N, 1), c1, c2, t1_w, t1_b.reshape(1, Dq),
      ln_g.reshape(1, Dq), ln_b.reshape(1, Dq), t2_w, t2_b.reshape(1, D))


def kernel(obj_feats, union_feats, rel_pair_idx, ws_w, ws_b, wo_w, wo_b,
           wu_w, wu_b, w_w, w_b, wt3_w, wt3_b, t1_w, t1_b, ln_g, ln_b,
           t2_w, t2_b):
    r0 = rel_pair_idx[:, 0].astype(jnp.int32)
    r1 = rel_pair_idx[:, 1].astype(jnp.int32)

    s_tab, og_tab, t3 = _prep(obj_feats, ws_w, ws_b, wo_w, wo_b, w_w,
                              wt3_w, wt3_b)

    sg, og = _sc_gather(s_tab, og_tab, r0, r1)

    coeff = _coeff(union_feats, sg, og, wu_w, wu_b, w_b)

    key = r0 * N + r1
    zf32 = jnp.zeros((_SLAB * N,), jnp.float32)
    zi32 = jnp.zeros((_CAPP,), jnp.int32)
    mi32 = jnp.full((_CAPP,), -1, jnp.int32)
    c1f, rowsum, eba, ed2, cnts = _sc_edges_a(key, coeff, t3, zf32, zi32,
                                              mi32)
    c2f = _sc_edges_b(eba, ed2, cnts, t3, zf32, zi32)
    c1 = c1f.reshape(N, DH)
    c2 = c2f.reshape(N, DH)

    return _mlp(obj_feats, t3, rowsum, c1, c2, t1_w, t1_b, ln_g, ln_b,
                t2_w, t2_b)
